# baseline (speedup 1.0000x reference)
import functools

import jax
import jax.numpy as jnp
from jax import lax
from jax.experimental import pallas as pl
from jax.experimental.pallas import tpu as pltpu
from jax.experimental.pallas import tpu_sc as plsc

_NC = 2
_NS = 16
_NW = _NC * _NS
_L = 16
_TS = 8
_TL = 128


@jax.jit
def _sc_gather_t(data, idx_t):
    n_j, b = idx_t.shape
    d = data.shape[1]
    bw = b // _NW
    kt = bw // _TL
    nc8 = d // _TS
    assert n_j % 2 == 0 and n_j >= 6 and bw % _TL == 0 and d % _TS == 0
    mesh = plsc.VectorSubcoreMesh(
        core_axis_name="c", subcore_axis_name="s",
        num_cores=_NC, num_subcores=_NS,
    )

    @functools.partial(
        pl.kernel,
        out_type=jax.ShapeDtypeStruct((n_j, nc8, b // _TL, _TS * _TL),
                                      data.dtype),
        mesh=mesh,
        scratch_types=[
            pltpu.VMEM((n_j, bw), jnp.int32),
            pltpu.VMEM((2, bw, d), data.dtype),
            pltpu.VMEM((2, nc8, kt, _TS * _TL), data.dtype),
            pltpu.SemaphoreType.DMA,
            pltpu.SemaphoreType.DMA((2,)),
            pltpu.SemaphoreType.DMA((2,)),
        ],
        compiler_params=pltpu.CompilerParams(use_tc_tiling_on_sc=False,
                                             needs_layout_passes=False),
    )
    def k(table_hbm, idx_hbm, out_hbm, idx_v, rows_v, tiles_v,
          isem, gsem, wsem):
        wid = lax.axis_index("s") * _NC + lax.axis_index("c")
        b0 = wid * bw

        pltpu.async_copy(idx_hbm.at[:, pl.ds(b0, bw)], idx_v, isem).wait()

        def start_gather(g, s):
            return pltpu.async_copy(
                table_hbm.at[idx_v.at[g]], rows_v.at[s], gsem.at[s])

        def wait_gather(s):
            pltpu.make_async_copy(
                table_hbm.at[idx_v.at[0]], rows_v.at[s], gsem.at[s]).wait()

        def start_write(g, s):
            return pltpu.async_copy(
                tiles_v.at[s], out_hbm.at[g, :, pl.ds(kt * wid, kt)],
                wsem.at[s])

        def wait_write(s):
            pltpu.make_async_copy(
                tiles_v.at[s], out_hbm.at[0, :, pl.ds(kt * wid, kt)],
                wsem.at[s]).wait()

        def transpose(s):
            rows = rows_v.at[s]
            def m_body(m, carry):
                row_base = lax.iota(jnp.int32, _L) + m * _L
                for i in range(nc8):
                    for kk in range(kt):
                        rvec = row_base + kk * _TL
                        for r in range(_TS):
                            cvec = jnp.full((_L,), i * _TS + r, jnp.int32)
                            vals = plsc.load_gather(rows, [rvec, cvec])
                            tiles_v[s, i, kk,
                                    pl.ds(r * _TL + m * _L, _L)] = vals
                return carry
            lax.fori_loop(0, _TL // _L, m_body, 0)

        def handler(g, s, issue_next, first):
            wait_gather(s)
            if not first:
                wait_write(s)
            transpose(s)
            if issue_next:
                start_gather(g + 2, s)
            start_write(g, s)

        start_gather(0, 0)
        start_gather(1, 1)
        handler(0, 0, True, True)
        handler(1, 1, True, True)

        def outer(t, carry):
            g = 2 + 2 * t
            handler(g, 0, True, False)
            handler(g + 1, 1, True, False)
            return carry

        lax.fori_loop(0, (n_j - 4) // 2, outer, 0)

        handler(n_j - 2, 0, False, False)
        handler(n_j - 1, 1, False, False)
        wait_write(0)
        wait_write(1)

    return k(data, idx_t)


def kernel(data, indices, axis):
    del axis
    b, n_j = indices.shape
    d = data.shape[1]
    out5 = _sc_gather_t(data, indices.T)
    out = out5.reshape(n_j, d // _TS, b // _TL, _TS, _TL)
    return out.transpose(2, 4, 0, 1, 3).reshape(b, n_j, d)


# speedup vs baseline: 1.0058x; 1.0058x over previous
import functools

import jax
import jax.numpy as jnp
from jax import lax
from jax.experimental import pallas as pl
from jax.experimental.pallas import tpu as pltpu
from jax.experimental.pallas import tpu_sc as plsc

_NC = 2
_NS = 16
_NW = _NC * _NS
_L = 16
_TS = 8
_TL = 128


@jax.jit
def _sc_gather_t(data, idx_t):
    n_j, b = idx_t.shape
    d = data.shape[1]
    bw = b // _NW
    kt = bw // _TL
    nc8 = d // _TS
    assert n_j % 2 == 0 and n_j >= 6 and bw % _TL == 0 and d % _TS == 0
    mesh = plsc.VectorSubcoreMesh(
        core_axis_name="c", subcore_axis_name="s",
        num_cores=_NC, num_subcores=_NS,
    )

    @functools.partial(
        pl.kernel,
        out_type=jax.ShapeDtypeStruct((n_j, nc8, b // _TL, _TS * _TL),
                                      data.dtype),
        mesh=mesh,
        scratch_types=[
            pltpu.VMEM((n_j, bw), jnp.int32),
            pltpu.VMEM((2, bw, d), data.dtype),
            pltpu.VMEM((2, nc8, kt, _TS * _TL), data.dtype),
            pltpu.SemaphoreType.DMA,
            pltpu.SemaphoreType.DMA((2,)),
            pltpu.SemaphoreType.DMA((2,)),
        ],
        compiler_params=pltpu.CompilerParams(use_tc_tiling_on_sc=False,
                                             needs_layout_passes=False),
    )
    def k(table_hbm, idx_hbm, out_hbm, idx_v, rows_v, tiles_v,
          isem, gsem, wsem):
        wid = lax.axis_index("s") * _NC + lax.axis_index("c")
        b0 = wid * bw

        pltpu.async_copy(idx_hbm.at[:, pl.ds(b0, bw)], idx_v, isem).wait()

        def start_gather(g, s):
            return pltpu.async_copy(
                table_hbm.at[idx_v.at[g]], rows_v.at[s], gsem.at[s])

        def wait_gather(s):
            pltpu.make_async_copy(
                table_hbm.at[idx_v.at[0]], rows_v.at[s], gsem.at[s]).wait()

        def start_write(g, s):
            return pltpu.async_copy(
                tiles_v.at[s], out_hbm.at[g, :, pl.ds(kt * wid, kt)],
                wsem.at[s])

        def wait_write(s):
            pltpu.make_async_copy(
                tiles_v.at[s], out_hbm.at[0, :, pl.ds(kt * wid, kt)],
                wsem.at[s]).wait()

        def transpose(s):
            rows = rows_v.at[s]
            def km_body(t, carry):
                kk = t // (_TL // _L)
                m = t % (_TL // _L)
                rvec = lax.iota(jnp.int32, _L) + (kk * _TL + m * _L)
                for i in range(nc8):
                    for r in range(_TS):
                        cvec = jnp.full((_L,), i * _TS + r, jnp.int32)
                        vals = plsc.load_gather(rows, [rvec, cvec])
                        tiles_v[s, i, kk, pl.ds(r * _TL + m * _L, _L)] = vals
                return carry
            lax.fori_loop(0, kt * (_TL // _L), km_body, 0)

        def handler(g, s, issue_next, first):
            wait_gather(s)
            if not first:
                wait_write(s)
            transpose(s)
            if issue_next:
                start_gather(g + 2, s)
            start_write(g, s)

        start_gather(0, 0)
        start_gather(1, 1)
        handler(0, 0, True, True)
        handler(1, 1, True, True)

        def outer(t, carry):
            g = 2 + 2 * t
            handler(g, 0, True, False)
            handler(g + 1, 1, True, False)
            return carry

        lax.fori_loop(0, (n_j - 4) // 2, outer, 0)

        handler(n_j - 2, 0, False, False)
        handler(n_j - 1, 1, False, False)
        wait_write(0)
        wait_write(1)

    return k(data, idx_t)


def kernel(data, indices, axis):
    del axis
    b, n_j = indices.shape
    d = data.shape[1]
    out5 = _sc_gather_t(data, indices.T)
    out = out5.reshape(n_j, d // _TS, b // _TL, _TS, _TL)
    return out.transpose(2, 4, 0, 1, 3).reshape(b, n_j, d)


# speedup vs baseline: 1.2017x; 1.1948x over previous
import functools

import jax
import jax.numpy as jnp
from jax import lax
from jax.experimental import pallas as pl
from jax.experimental.pallas import tpu as pltpu
from jax.experimental.pallas import tpu_sc as plsc

_NC = 2
_NS = 16
_NW = _NC * _NS

_NBUF = 5


@jax.jit
def _sc_gather(data, idx_t):
    n_j, b = idx_t.shape
    d = data.shape[1]
    bw = b // _NW
    assert n_j % _NBUF == 0 and n_j >= 2 * _NBUF
    mesh = plsc.VectorSubcoreMesh(
        core_axis_name="c", subcore_axis_name="s",
        num_cores=_NC, num_subcores=_NS,
    )

    @functools.partial(
        pl.kernel,
        out_type=jax.ShapeDtypeStruct((n_j, b, d), data.dtype),
        mesh=mesh,
        scratch_types=[
            pltpu.VMEM((n_j, bw), jnp.int32),
            pltpu.VMEM((_NBUF, bw, d), data.dtype),
            pltpu.SemaphoreType.DMA,
            pltpu.SemaphoreType.DMA((_NBUF,)),
            pltpu.SemaphoreType.DMA((_NBUF,)),
        ],
        compiler_params=pltpu.CompilerParams(use_tc_tiling_on_sc=False),
    )
    def k(table_hbm, idx_hbm, out_hbm, idx_v, rows_v, isem, gsem, wsem):
        wid = lax.axis_index("s") * _NC + lax.axis_index("c")
        b0 = wid * bw

        pltpu.async_copy(
            idx_hbm.at[:, pl.ds(b0, bw)], idx_v, isem).wait()

        def start_gather(g, slot):
            return pltpu.async_copy(
                table_hbm.at[idx_v.at[g]], rows_v.at[slot], gsem.at[slot])

        def start_write(g, slot):
            return pltpu.async_copy(
                rows_v.at[slot], out_hbm.at[g, pl.ds(b0, bw)],
                wsem.at[slot])

        def wait_gather(slot):
            pltpu.make_async_copy(
                table_hbm.at[idx_v.at[0]], rows_v.at[slot],
                gsem.at[slot]).wait()

        def wait_write(slot):
            pltpu.make_async_copy(
                rows_v.at[slot], out_hbm.at[0, pl.ds(b0, bw)],
                wsem.at[slot]).wait()

        start_gather(0, 0)
        for g in range(1, _NBUF):
            start_gather(g, g)
            wait_gather(g - 1)
            start_write(g - 1, g - 1)

        def outer(t, carry):
            g0 = _NBUF + t * _NBUF
            for s in range(_NBUF):
                g = g0 + s
                wait_write(s)
                start_gather(g, s)
                sp = (s - 1) % _NBUF
                wait_gather(sp)
                start_write(g - 1, sp)
            return carry

        lax.fori_loop(0, n_j // _NBUF - 1, outer, 0)

        last_s = (n_j - 1) % _NBUF
        wait_gather(last_s)
        start_write(n_j - 1, last_s)
        for s in range(_NBUF):
            wait_write(s)

    return k(data, idx_t)


def kernel(data, indices, axis):
    del axis
    d = data.shape[1]
    table = jnp.pad(data, ((0, 0), (0, 128 - d))).reshape(-1, d)
    out = _sc_gather(table, indices.T * (128 // d))
    return out.transpose(1, 0, 2)
